# Initial kernel scaffold; baseline (speedup 1.0000x reference)
import jax
import jax.numpy as jnp
from jax.experimental import pallas as pl


def kernel(embedding, triplets, w_relation):
    raise NotImplementedError("write your pallas kernel here")


# speedup vs baseline: 1.1628x; 1.1628x over previous
import functools

import jax
import jax.numpy as jnp
from jax import lax
from jax.experimental import pallas as pl
from jax.experimental.pallas import tpu as pltpu
from jax.experimental.pallas import tpu_sc as plsc

N_TRIPLETS = 320000
D = 128
NC = 2
NS = 16
NW = NC * NS
PW = N_TRIPLETS // NW
C = 80
K = PW // C
LANES = 16
GROUPS = D // LANES


def _sc_body(emb_hbm, wrel_hbm, sidx_hbm, ridx_hbm, oidx_hbm, out_hbm,
             sidx_v, ridx_v, oidx_v, srow_v, rrow_v, orow_v, scores_v, sem):
    wid = lax.axis_index("s") * NC + lax.axis_index("c")
    base = wid * PW

    pltpu.sync_copy(sidx_hbm.at[pl.ds(base, PW)], sidx_v)
    pltpu.sync_copy(ridx_hbm.at[pl.ds(base, PW)], ridx_v)
    pltpu.sync_copy(oidx_hbm.at[pl.ds(base, PW)], oidx_v)

    def chunk_body(c, carry):
        off = c * C
        cs = pltpu.async_copy(emb_hbm.at[sidx_v.at[pl.ds(off, C)]], srow_v, sem)
        cr = pltpu.async_copy(wrel_hbm.at[ridx_v.at[pl.ds(off, C)]], rrow_v, sem)
        co = pltpu.async_copy(emb_hbm.at[oidx_v.at[pl.ds(off, C)]], orow_v, sem)
        cs.wait()
        cr.wait()
        co.wait()

        iota16 = lax.iota(jnp.int32, LANES)
        for t in range(C // LANES):
            rows = t * LANES + iota16

            def dbody(dd, acc):
                dv = jnp.full((LANES,), dd, jnp.int32)
                sv = plsc.load_gather(srow_v, [rows, dv])
                rv = plsc.load_gather(rrow_v, [rows, dv])
                ov = plsc.load_gather(orow_v, [rows, dv])
                return acc + sv * rv * ov

            acc = lax.fori_loop(0, D, dbody, jnp.zeros((LANES,), jnp.float32),
                                unroll=4)
            scores_v[pl.ds(off + t * LANES, LANES)] = acc
        return carry

    lax.fori_loop(0, K, chunk_body, 0)

    pltpu.sync_copy(scores_v, out_hbm.at[pl.ds(base, PW)])


@jax.jit
def kernel(embedding, triplets, w_relation):
    trip = triplets.astype(jnp.int32)
    s_idx = trip[:, 0]
    r_idx = trip[:, 1]
    o_idx = trip[:, 2]

    mesh = plsc.VectorSubcoreMesh(core_axis_name="c", subcore_axis_name="s")
    k = pl.kernel(
        _sc_body,
        out_type=jax.ShapeDtypeStruct((N_TRIPLETS,), jnp.float32),
        mesh=mesh,
        compiler_params=pltpu.CompilerParams(needs_layout_passes=False),
        scratch_types=[
            pltpu.VMEM((PW,), jnp.int32),
            pltpu.VMEM((PW,), jnp.int32),
            pltpu.VMEM((PW,), jnp.int32),
            pltpu.VMEM((C, D), jnp.float32),
            pltpu.VMEM((C, D), jnp.float32),
            pltpu.VMEM((C, D), jnp.float32),
            pltpu.VMEM((PW,), jnp.float32),
            pltpu.SemaphoreType.DMA,
        ],
    )
    return k(embedding, w_relation, s_idx, r_idx, o_idx)


# speedup vs baseline: 9.8586x; 8.4786x over previous
import jax
import jax.numpy as jnp
from jax import lax
from jax.experimental import pallas as pl
from jax.experimental.pallas import tpu as pltpu
from jax.experimental.pallas import tpu_sc as plsc

N_TRIPLETS = 320000
D = 128
NC = 2
NS = 16
NW = NC * NS
PW = N_TRIPLETS // NW
C = 80
K = PW // C
LANES = 16


def _sc_body(emb_hbm, wrel_hbm, sidx_hbm, ridx_hbm, oidx_hbm, out_hbm,
             sidx_v, ridx_v, oidx_v,
             sbuf0, rbuf0, obuf0, sbuf1, rbuf1, obuf1, scores_v,
             sem0, sem1):
    wid = lax.axis_index("s") * NC + lax.axis_index("c")
    base = wid * PW

    pltpu.sync_copy(sidx_hbm.at[pl.ds(base, PW)], sidx_v)
    pltpu.sync_copy(ridx_hbm.at[pl.ds(base, PW)], ridx_v)
    pltpu.sync_copy(oidx_hbm.at[pl.ds(base, PW)], oidx_v)

    bufs = ((sbuf0, rbuf0, obuf0, sem0), (sbuf1, rbuf1, obuf1, sem1))

    def issue(c, slot):
        sb, rb, ob, sem = bufs[slot]
        off = c * C
        pltpu.async_copy(emb_hbm.at[sidx_v.at[pl.ds(off, C)]], sb, sem)
        pltpu.async_copy(wrel_hbm.at[ridx_v.at[pl.ds(off, C)]], rb, sem)
        pltpu.async_copy(emb_hbm.at[oidx_v.at[pl.ds(off, C)]], ob, sem)

    def wait(c, slot):
        sb, rb, ob, sem = bufs[slot]
        off = c * C
        pltpu.make_async_copy(emb_hbm.at[sidx_v.at[pl.ds(off, C)]], sb, sem).wait()
        pltpu.make_async_copy(wrel_hbm.at[ridx_v.at[pl.ds(off, C)]], rb, sem).wait()
        pltpu.make_async_copy(emb_hbm.at[oidx_v.at[pl.ds(off, C)]], ob, sem).wait()

    iota16 = lax.iota(jnp.int32, LANES)

    def compute(c, slot):
        sb, rb, ob, _ = bufs[slot]
        off = c * C
        for t in range(C // LANES):
            rows = t * LANES + iota16

            def dbody(dd, acc):
                cols = (iota16 + dd) & (D - 1)
                sv = plsc.load_gather(sb, [rows, cols])
                rv = plsc.load_gather(rb, [rows, cols])
                ov = plsc.load_gather(ob, [rows, cols])
                return acc + sv * rv * ov

            acc = lax.fori_loop(0, D, dbody, jnp.zeros((LANES,), jnp.float32),
                                unroll=8)
            scores_v[pl.ds(off + t * LANES, LANES)] = acc

    issue(0, 0)

    def pair_body(p, carry):
        c0 = 2 * p
        issue(c0 + 1, 1)
        wait(c0, 0)
        compute(c0, 0)
        issue(c0 + 2, 0)
        wait(c0 + 1, 1)
        compute(c0 + 1, 1)
        return carry

    lax.fori_loop(0, (K - 1) // 2, pair_body, 0)
    wait(K - 1, 0)
    compute(K - 1, 0)

    pltpu.sync_copy(scores_v, out_hbm.at[pl.ds(base, PW)])


@jax.jit
def kernel(embedding, triplets, w_relation):
    trip = triplets.astype(jnp.int32)
    s_idx = trip[:, 0]
    r_idx = trip[:, 1]
    o_idx = trip[:, 2]

    mesh = plsc.VectorSubcoreMesh(core_axis_name="c", subcore_axis_name="s")
    row_buf = pltpu.VMEM((C, D), jnp.float32)
    k = pl.kernel(
        _sc_body,
        out_type=jax.ShapeDtypeStruct((N_TRIPLETS,), jnp.float32),
        mesh=mesh,
        compiler_params=pltpu.CompilerParams(needs_layout_passes=False),
        scratch_types=[
            pltpu.VMEM((PW,), jnp.int32),
            pltpu.VMEM((PW,), jnp.int32),
            pltpu.VMEM((PW,), jnp.int32),
            row_buf, row_buf, row_buf,
            row_buf, row_buf, row_buf,
            pltpu.VMEM((PW,), jnp.float32),
            pltpu.SemaphoreType.DMA,
            pltpu.SemaphoreType.DMA,
        ],
    )
    return k(embedding, w_relation, s_idx, r_idx, o_idx)
